# baseline (speedup 1.0000x reference)
import jax
import jax.numpy as jnp
from jax.experimental import pallas as pl
from jax.experimental.pallas import tpu as pltpu

ROW_BLK = 256
N_ROW = 4096
N_COL = 2048
N_BATCH = 4
N_K = N_ROW // ROW_BLK


def _argmin_body(x_ref, o_ref, mval, midx):
    k = pl.program_id(1)
    xb = x_ref[0]
    m = jnp.min(xb, axis=0, keepdims=True)
    rows = jax.lax.broadcasted_iota(jnp.int32, (ROW_BLK, N_COL), 0) + k * ROW_BLK
    im = jnp.min(jnp.where(xb == m, rows, jnp.int32(2**30)), axis=0, keepdims=True)

    @pl.when(k == 0)
    def _init():
        mval[...] = m
        midx[...] = im

    @pl.when(k > 0)
    def _merge():
        better = m < mval[...]
        mval[...] = jnp.where(better, m, mval[...])
        midx[...] = jnp.where(better, im, midx[...])

    @pl.when(k == N_K - 1)
    def _emit():
        o_ref[0] = midx[...]


def kernel(x):
    out = pl.pallas_call(
        _argmin_body,
        grid=(N_BATCH, N_K),
        in_specs=[pl.BlockSpec((1, ROW_BLK, N_COL), lambda b, k: (b, k, 0))],
        out_specs=pl.BlockSpec((1, 1, N_COL), lambda b, k: (b, 0, 0)),
        out_shape=jax.ShapeDtypeStruct((N_BATCH, 1, N_COL), jnp.int32),
        scratch_shapes=[
            pltpu.VMEM((1, N_COL), jnp.float32),
            pltpu.VMEM((1, N_COL), jnp.int32),
        ],
    )(x)
    return out.reshape(N_BATCH, N_COL).astype(jnp.int64)
